# baseline (speedup 1.0000x reference)
import functools

import jax
import jax.numpy as jnp
from jax import lax
from jax.experimental import pallas as pl
from jax.experimental.pallas import tpu as pltpu
from jax.experimental.pallas import tpu_sc as plsc

VN_DEG = 3
CN_DEG = 6
NUM_ITER = 20
LLR_CLIP = 20.0
PHI_MIN = 8.5e-8
PHI_MAX = 16.635532

_SC_CORES = 2
_SC_SUBCORES = 16
_SC_WORKERS = _SC_CORES * _SC_SUBCORES
_IDX_CHUNK = 128


def _phi(x):
    x = jnp.clip(x, PHI_MIN, PHI_MAX)
    return -jnp.log(jnp.tanh(0.5 * x))


def _cn_body(m_ref, o_ref):
    m = m_ref[...]
    s = jnp.where(m < 0.0, -1.0, 1.0)
    p = _phi(jnp.abs(m))
    ptot = ((p[0] + p[1]) + (p[2] + p[3])) + (p[4] + p[5])
    stot = ((s[0] * s[1]) * (s[2] * s[3])) * (s[4] * s[5])
    o_ref[...] = (stot[None] * s) * _phi(ptot[None] - p)


def _vn_body(mc_ref, llr_ref, msg_ref, marg_ref):
    mc = mc_ref[...]
    llr = llr_ref[...]
    tot = llr + (mc[0] + mc[1] + mc[2])
    marg_ref[...] = tot
    msg_ref[...] = jnp.clip(tot[None] - mc, -LLR_CLIP, LLR_CLIP)


def _cn_stage(msg_c, n_cns, batch, block):
    grid = n_cns // block
    return pl.pallas_call(
        _cn_body,
        grid=(grid,),
        in_specs=[pl.BlockSpec((CN_DEG, block, batch), lambda i: (0, i, 0))],
        out_specs=pl.BlockSpec((CN_DEG, block, batch), lambda i: (0, i, 0)),
        out_shape=jax.ShapeDtypeStruct((CN_DEG, n_cns, batch), jnp.float32),
    )(msg_c)


def _vn_stage(msg_cn_v, llr, n_vns, batch, block):
    grid = n_vns // block
    return pl.pallas_call(
        _vn_body,
        grid=(grid,),
        in_specs=[
            pl.BlockSpec((VN_DEG, block, batch), lambda i: (0, i, 0)),
            pl.BlockSpec((block, batch), lambda i: (i, 0)),
        ],
        out_specs=[
            pl.BlockSpec((VN_DEG, block, batch), lambda i: (0, i, 0)),
            pl.BlockSpec((block, batch), lambda i: (i, 0)),
        ],
        out_shape=[
            jax.ShapeDtypeStruct((VN_DEG, n_vns, batch), jnp.float32),
            jax.ShapeDtypeStruct((n_vns, batch), jnp.float32),
        ],
    )(msg_cn_v, llr)


def _sc_gather(src, idx, n_out, batch):
    rows_w = n_out // _SC_WORKERS
    n_ch = rows_w // _IDX_CHUNK
    mesh = plsc.VectorSubcoreMesh(core_axis_name="c", subcore_axis_name="s")

    @functools.partial(
        pl.kernel,
        out_type=jax.ShapeDtypeStruct((n_out, batch), jnp.float32),
        mesh=mesh,
        scratch_types=[
            pltpu.VMEM((rows_w,), jnp.int32),
            pltpu.VMEM((rows_w, batch), jnp.float32),
            pltpu.SemaphoreType.DMA,
        ],
    )
    def k(src_hbm, idx_hbm, out_hbm, idx_v, rows_v, sem):
        wid = lax.axis_index("s") * _SC_CORES + lax.axis_index("c")
        pltpu.sync_copy(idx_hbm.at[pl.ds(wid * rows_w, rows_w)], idx_v)
        copies = []
        for j in range(n_ch):
            copies.append(pltpu.async_copy(
                src_hbm.at[idx_v.at[pl.ds(j * _IDX_CHUNK, _IDX_CHUNK)]],
                rows_v.at[pl.ds(j * _IDX_CHUNK, _IDX_CHUNK)],
                sem,
            ))
        for c in copies:
            c.wait()
        pltpu.sync_copy(rows_v, out_hbm.at[pl.ds(wid * rows_w, rows_w)])

    return k(src, idx)


def _sc_vn_fused(msg_cn_c, llr, idx, n_edges, n_vns, batch, with_marg=True):
    vpw = n_vns // _SC_WORKERS
    ch = 64
    n_ch = vpw // ch
    nsets = 3
    cpi = VN_DEG * ch
    mesh = plsc.VectorSubcoreMesh(core_axis_name="c", subcore_axis_name="s")

    out_type = [jax.ShapeDtypeStruct((n_edges, batch), jnp.float32)]
    if with_marg:
        out_type.append(jax.ShapeDtypeStruct((n_vns, batch), jnp.float32))

    @functools.partial(
        pl.kernel,
        out_type=out_type,
        mesh=mesh,
        scratch_types=(
            [pltpu.VMEM((n_ch * cpi,), jnp.int32),
             pltpu.VMEM((n_ch * VN_DEG, ch), jnp.int32),
             pltpu.VMEM((vpw, batch), jnp.float32)]
            + [pltpu.VMEM((2 * ch, batch), jnp.float32),
               pltpu.VMEM((ch, batch), jnp.float32)] * nsets
            + [pltpu.SemaphoreType.DMA] * (2 * nsets + 1)
        ),
    )
    def k(mcn, llr_h, idx_h, idx3_h, out_msg, *rest2):
        if with_marg:
            out_marg, idx_f, idx_v, ll = rest2[:4]
        else:
            idx_f, idx_v, ll = rest2[:3]
        rest = rest2[4 if with_marg else 3:]
        bufs = rest[:2 * nsets]
        sem_g = rest[2 * nsets:3 * nsets]
        sem_s = rest[3 * nsets:4 * nsets]
        sem_l = rest[4 * nsets]
        wid = lax.axis_index("s") * _SC_CORES + lax.axis_index("c")
        ibase = wid * (n_ch * cpi)
        pltpu.sync_copy(idx_h.at[pl.ds(ibase, n_ch * cpi)], idx_f)
        dl = pltpu.async_copy(llr_h.at[pl.ds(wid * vpw, vpw)], ll, sem_l)
        pltpu.sync_copy(idx3_h.at[wid], idx_v)

        gat, scat = {}, {}

        def fire_gather(c):
            s = c % nsets
            ab, g2 = bufs[2 * s:2 * s + 2]
            gat[c] = [
                pltpu.async_copy(
                    mcn.at[idx_f.at[pl.ds(c * cpi, 2 * ch)]], ab, sem_g[s]),
                pltpu.async_copy(
                    mcn.at[idx_f.at[pl.ds(c * cpi + 2 * ch, ch)]], g2,
                    sem_g[s]),
            ]

        def fire_scatter(c):
            s = c % nsets
            ab, g2 = bufs[2 * s:2 * s + 2]
            scat[c] = [
                pltpu.async_copy(ab.at[pl.ds(0, ch)],
                                 out_msg.at[idx_v.at[c * VN_DEG]], sem_s[s]),
                pltpu.async_copy(ab.at[pl.ds(ch, ch)],
                                 out_msg.at[idx_v.at[c * VN_DEG + 1]],
                                 sem_s[s]),
                pltpu.async_copy(g2,
                                 out_msg.at[idx_v.at[c * VN_DEG + 2]],
                                 sem_s[s]),
            ]

        def compute(c):
            s = c % nsets
            ab, g2 = bufs[2 * s:2 * s + 2]

            def row(r, carry):
                lr = c * ch + r
                for u in range(batch // 16):
                    sl = pl.ds(u * 16, 16)
                    a, b = ab[r, sl], ab[ch + r, sl]
                    g, l = g2[r, sl], ll[lr, sl]
                    tot = l + ((a + b) + g)
                    ll[lr, sl] = tot
                    ab[r, sl] = jnp.clip(tot - a, -LLR_CLIP, LLR_CLIP)
                    ab[ch + r, sl] = jnp.clip(tot - b, -LLR_CLIP, LLR_CLIP)
                    g2[r, sl] = jnp.clip(tot - g, -LLR_CLIP, LLR_CLIP)
                return carry

            lax.fori_loop(0, ch, row, 0)

        fire_gather(0)
        if n_ch > 1:
            fire_gather(1)
        dl.wait()
        for c in range(n_ch):
            for d in gat[c]:
                d.wait()
            compute(c)
            fire_scatter(c)
            nxt = c + 2
            if nxt < n_ch:
                prev = nxt - nsets
                if prev >= 0:
                    for d in scat[prev]:
                        d.wait()
                fire_gather(nxt)
        for c in range(max(0, n_ch - nsets), n_ch):
            for d in scat[c]:
                d.wait()
        pltpu.sync_copy(ll, out_marg.at[pl.ds(wid * vpw, vpw)])

    idx3 = idx.reshape(_SC_WORKERS, n_ch * VN_DEG, ch)
    return k(msg_cn_c, llr, idx, idx3)


def kernel(llrs_ch, vn_idx, cn_idx):
    batch, n_vns = llrs_ch.shape
    n_edges = vn_idx.shape[0]
    n_cns = n_edges // CN_DEG

    order = jnp.argsort(cn_idx).astype(jnp.int32)
    a = (order % VN_DEG) * n_vns + order // VN_DEG
    g_vc = a.reshape(n_cns, CN_DEG).T.reshape(-1)
    sigma = jnp.argsort(a).astype(jnp.int32)
    g_cv = (sigma % CN_DEG) * n_cns + sigma // CN_DEG
    init_idx = g_vc % n_vns
    vpw = n_vns // _SC_WORKERS
    idx_fused = (g_cv.reshape(VN_DEG, n_vns)
                 .reshape(VN_DEG, _SC_WORKERS, vpw // 64, 64)
                 .transpose(1, 2, 0, 3).reshape(-1))

    llr = jnp.clip(llrs_ch, -LLR_CLIP, LLR_CLIP).T

    msg_c0 = _sc_gather(llr, init_idx, n_edges, batch)
    msg_c0 = msg_c0.reshape(CN_DEG, n_cns, batch)

    def body(_, carry):
        msg_c, _marg = carry
        msg_cn_c = _cn_stage(msg_c, n_cns, batch, block=512)
        msg_c_next, marg = _sc_vn_fused(
            msg_cn_c.reshape(n_edges, batch), llr, idx_fused,
            n_edges, n_vns, batch)
        return msg_c_next.reshape(CN_DEG, n_cns, batch), marg

    _, marg = lax.fori_loop(
        0, NUM_ITER, body, (msg_c0, jnp.zeros_like(llr)))
    return marg.T


# speedup vs baseline: 1.0412x; 1.0412x over previous
import functools

import jax
import jax.numpy as jnp
from jax import lax
from jax.experimental import pallas as pl
from jax.experimental.pallas import tpu as pltpu
from jax.experimental.pallas import tpu_sc as plsc

VN_DEG = 3
CN_DEG = 6
NUM_ITER = 20
LLR_CLIP = 20.0
PHI_MIN = 8.5e-8
PHI_MAX = 16.635532

_SC_CORES = 2
_SC_SUBCORES = 16
_SC_WORKERS = _SC_CORES * _SC_SUBCORES
_IDX_CHUNK = 128


def _phi(x):
    x = jnp.clip(x, PHI_MIN, PHI_MAX)
    return -jnp.log(jnp.tanh(0.5 * x))


def _cn_body(m_ref, o_ref):
    m = m_ref[...]
    s = jnp.where(m < 0.0, -1.0, 1.0)
    p = _phi(jnp.abs(m))
    ptot = ((p[0] + p[1]) + (p[2] + p[3])) + (p[4] + p[5])
    stot = ((s[0] * s[1]) * (s[2] * s[3])) * (s[4] * s[5])
    o_ref[...] = (stot[None] * s) * _phi(ptot[None] - p)


def _vn_body(mc_ref, llr_ref, msg_ref, marg_ref):
    mc = mc_ref[...]
    llr = llr_ref[...]
    tot = llr + (mc[0] + mc[1] + mc[2])
    marg_ref[...] = tot
    msg_ref[...] = jnp.clip(tot[None] - mc, -LLR_CLIP, LLR_CLIP)


def _cn_stage(msg_c, n_cns, batch, block):
    grid = n_cns // block
    return pl.pallas_call(
        _cn_body,
        grid=(grid,),
        in_specs=[pl.BlockSpec((CN_DEG, block, batch), lambda i: (0, i, 0))],
        out_specs=pl.BlockSpec((CN_DEG, block, batch), lambda i: (0, i, 0)),
        out_shape=jax.ShapeDtypeStruct((CN_DEG, n_cns, batch), jnp.float32),
    )(msg_c)


def _vn_stage(msg_cn_v, llr, n_vns, batch, block):
    grid = n_vns // block
    return pl.pallas_call(
        _vn_body,
        grid=(grid,),
        in_specs=[
            pl.BlockSpec((VN_DEG, block, batch), lambda i: (0, i, 0)),
            pl.BlockSpec((block, batch), lambda i: (i, 0)),
        ],
        out_specs=[
            pl.BlockSpec((VN_DEG, block, batch), lambda i: (0, i, 0)),
            pl.BlockSpec((block, batch), lambda i: (i, 0)),
        ],
        out_shape=[
            jax.ShapeDtypeStruct((VN_DEG, n_vns, batch), jnp.float32),
            jax.ShapeDtypeStruct((n_vns, batch), jnp.float32),
        ],
    )(msg_cn_v, llr)


def _sc_gather(src, idx, n_out, batch):
    rows_w = n_out // _SC_WORKERS
    n_ch = rows_w // _IDX_CHUNK
    mesh = plsc.VectorSubcoreMesh(core_axis_name="c", subcore_axis_name="s")

    @functools.partial(
        pl.kernel,
        out_type=jax.ShapeDtypeStruct((n_out, batch), jnp.float32),
        mesh=mesh,
        scratch_types=[
            pltpu.VMEM((rows_w,), jnp.int32),
            pltpu.VMEM((rows_w, batch), jnp.float32),
            pltpu.SemaphoreType.DMA,
        ],
    )
    def k(src_hbm, idx_hbm, out_hbm, idx_v, rows_v, sem):
        wid = lax.axis_index("s") * _SC_CORES + lax.axis_index("c")
        pltpu.sync_copy(idx_hbm.at[pl.ds(wid * rows_w, rows_w)], idx_v)
        copies = []
        for j in range(n_ch):
            copies.append(pltpu.async_copy(
                src_hbm.at[idx_v.at[pl.ds(j * _IDX_CHUNK, _IDX_CHUNK)]],
                rows_v.at[pl.ds(j * _IDX_CHUNK, _IDX_CHUNK)],
                sem,
            ))
        for c in copies:
            c.wait()
        pltpu.sync_copy(rows_v, out_hbm.at[pl.ds(wid * rows_w, rows_w)])

    return k(src, idx)


def _sc_vn_fused(msg_cn_c, llr, idx, n_edges, n_vns, batch, with_marg=True):
    vpw = n_vns // _SC_WORKERS
    ch = 64
    n_ch = vpw // ch
    nsets = 3
    cpi = VN_DEG * ch
    mesh = plsc.VectorSubcoreMesh(core_axis_name="c", subcore_axis_name="s")

    out_type = [jax.ShapeDtypeStruct((n_edges, batch), jnp.float32)]
    if with_marg:
        out_type.append(jax.ShapeDtypeStruct((n_vns, batch), jnp.float32))

    @functools.partial(
        pl.kernel,
        out_type=out_type,
        mesh=mesh,
        scratch_types=(
            [pltpu.VMEM((n_ch * cpi,), jnp.int32),
             pltpu.VMEM((n_ch * VN_DEG, ch), jnp.int32),
             pltpu.VMEM((vpw, batch), jnp.float32)]
            + [pltpu.VMEM((2 * ch, batch), jnp.float32),
               pltpu.VMEM((ch, batch), jnp.float32)] * nsets
            + [pltpu.SemaphoreType.DMA] * (2 * nsets + 1)
        ),
    )
    def k(mcn, llr_h, idx_h, idx3_h, out_msg, *rest2):
        if with_marg:
            out_marg, idx_f, idx_v, ll = rest2[:4]
        else:
            idx_f, idx_v, ll = rest2[:3]
        rest = rest2[4 if with_marg else 3:]
        bufs = rest[:2 * nsets]
        sem_g = rest[2 * nsets:3 * nsets]
        sem_s = rest[3 * nsets:4 * nsets]
        sem_l = rest[4 * nsets]
        wid = lax.axis_index("s") * _SC_CORES + lax.axis_index("c")
        ibase = wid * (n_ch * cpi)
        pltpu.sync_copy(idx_h.at[pl.ds(ibase, n_ch * cpi)], idx_f)
        dl = pltpu.async_copy(llr_h.at[pl.ds(wid * vpw, vpw)], ll, sem_l)
        pltpu.sync_copy(idx3_h.at[wid], idx_v)

        gat, scat = {}, {}

        def fire_gather(c):
            s = c % nsets
            ab, g2 = bufs[2 * s:2 * s + 2]
            gat[c] = [
                pltpu.async_copy(
                    mcn.at[idx_f.at[pl.ds(c * cpi, 2 * ch)]], ab, sem_g[s]),
                pltpu.async_copy(
                    mcn.at[idx_f.at[pl.ds(c * cpi + 2 * ch, ch)]], g2,
                    sem_g[s]),
            ]

        def fire_scatter(c):
            s = c % nsets
            ab, g2 = bufs[2 * s:2 * s + 2]
            scat[c] = [
                pltpu.async_copy(ab.at[pl.ds(0, ch)],
                                 out_msg.at[idx_v.at[c * VN_DEG]], sem_s[s]),
                pltpu.async_copy(ab.at[pl.ds(ch, ch)],
                                 out_msg.at[idx_v.at[c * VN_DEG + 1]],
                                 sem_s[s]),
                pltpu.async_copy(g2,
                                 out_msg.at[idx_v.at[c * VN_DEG + 2]],
                                 sem_s[s]),
            ]

        def compute(c):
            s = c % nsets
            ab, g2 = bufs[2 * s:2 * s + 2]

            def row(r, carry):
                lr = c * ch + r
                for u in range(batch // 16):
                    sl = pl.ds(u * 16, 16)
                    a, b = ab[r, sl], ab[ch + r, sl]
                    g, l = g2[r, sl], ll[lr, sl]
                    tot = l + ((a + b) + g)
                    ll[lr, sl] = tot
                    ab[r, sl] = jnp.clip(tot - a, -LLR_CLIP, LLR_CLIP)
                    ab[ch + r, sl] = jnp.clip(tot - b, -LLR_CLIP, LLR_CLIP)
                    g2[r, sl] = jnp.clip(tot - g, -LLR_CLIP, LLR_CLIP)
                return carry

            lax.fori_loop(0, ch, row, 0)

        fire_gather(0)
        if n_ch > 1:
            fire_gather(1)
        dl.wait()
        for c in range(n_ch):
            for d in gat[c]:
                d.wait()
            compute(c)
            fire_scatter(c)
            nxt = c + 2
            if nxt < n_ch:
                prev = nxt - nsets
                if prev >= 0:
                    for d in scat[prev]:
                        d.wait()
                fire_gather(nxt)
        for c in range(max(0, n_ch - nsets), n_ch):
            for d in scat[c]:
                d.wait()
        if with_marg:
            pltpu.sync_copy(ll, out_marg.at[pl.ds(wid * vpw, vpw)])

    idx3 = idx.reshape(_SC_WORKERS, n_ch * VN_DEG, ch)
    return k(msg_cn_c, llr, idx, idx3)


def kernel(llrs_ch, vn_idx, cn_idx):
    batch, n_vns = llrs_ch.shape
    n_edges = vn_idx.shape[0]
    n_cns = n_edges // CN_DEG

    order = jnp.argsort(cn_idx).astype(jnp.int32)
    a = (order % VN_DEG) * n_vns + order // VN_DEG
    g_vc = a.reshape(n_cns, CN_DEG).T.reshape(-1)
    sigma = jnp.argsort(a).astype(jnp.int32)
    g_cv = (sigma % CN_DEG) * n_cns + sigma // CN_DEG
    init_idx = g_vc % n_vns
    vpw = n_vns // _SC_WORKERS
    idx_fused = (g_cv.reshape(VN_DEG, n_vns)
                 .reshape(VN_DEG, _SC_WORKERS, vpw // 64, 64)
                 .transpose(1, 2, 0, 3).reshape(-1))

    llr = jnp.clip(llrs_ch, -LLR_CLIP, LLR_CLIP).T

    msg_c0 = _sc_gather(llr, init_idx, n_edges, batch)
    msg_c0 = msg_c0.reshape(CN_DEG, n_cns, batch)

    def body(_, msg_c):
        msg_cn_c = _cn_stage(msg_c, n_cns, batch, block=512)
        (msg_c_next,) = _sc_vn_fused(
            msg_cn_c.reshape(n_edges, batch), llr, idx_fused,
            n_edges, n_vns, batch, with_marg=False)
        return msg_c_next.reshape(CN_DEG, n_cns, batch)

    msg_c = lax.fori_loop(0, NUM_ITER - 1, body, msg_c0)
    msg_cn_c = _cn_stage(msg_c, n_cns, batch, block=512)
    _, marg = _sc_vn_fused(
        msg_cn_c.reshape(n_edges, batch), llr, idx_fused,
        n_edges, n_vns, batch, with_marg=True)
    return marg.T
